# baseline (speedup 1.0000x reference)
import functools

import jax
import jax.numpy as jnp
from jax.experimental import pallas as pl
from jax.experimental.pallas import tpu as pltpu

_NUM_CODES = 8192
_CODE_DIM = 32
_BETA = 0.25
_B = 8192
_BR = 256
_NB = _B // _BR


def _vq_body(h_ref, cbt_ref, cb_ref, g_ref,
             q_ref, ct_ref, ch_ref, cq_ref, loss_ref):
    i = pl.program_id(0)
    h = h_ref[...]
    cbt = cbt_ref[...]
    cb_sq = jnp.sum(cbt * cbt, axis=0, keepdims=True)
    h_sq = jnp.sum(h * h, axis=1, keepdims=True)
    prod = jnp.dot(h, cbt, preferred_element_type=jnp.float32)
    dist = (h_sq + cb_sq) - 2.0 * prod
    x = g_ref[...] - dist
    m = jnp.max(x, axis=1, keepdims=True)
    e = jnp.exp(x - m)
    s = jnp.sum(e, axis=1, keepdims=True)
    inv = 1.0 / s
    q = e * inv
    q_ref[...] = q

    cb16 = cb_ref[...].astype(jnp.bfloat16)
    ct = jnp.dot(q.astype(jnp.bfloat16), cb16,
                 preferred_element_type=jnp.float32)
    ct_ref[...] = ct

    iota = jax.lax.broadcasted_iota(jnp.int32, (_BR, _NUM_CODES), 1)
    idx = jnp.min(jnp.where(x == m, iota, _NUM_CODES), axis=1, keepdims=True)
    onehot = (iota == idx).astype(jnp.bfloat16)
    ch = jnp.dot(onehot, cb16, preferred_element_type=jnp.float32)
    ch_ref[...] = ch
    cq_ref[...] = ct + (ch - ct)
    hh = h

    d = hh - ch
    part = jnp.sum(d * d, axis=(0, 1), keepdims=True)

    @pl.when(i == 0)
    def _init():
        loss_ref[...] = part

    @pl.when(i > 0)
    def _acc():
        loss_ref[...] += part


@jax.jit
def kernel(h, codebook, gumbel):
    cbt = codebook.T
    q, ct, ch, cq, loss_acc = pl.pallas_call(
        _vq_body,
        grid=(_NB,),
        in_specs=[
            pl.BlockSpec((_BR, _CODE_DIM), lambda i: (i, 0)),
            pl.BlockSpec((_CODE_DIM, _NUM_CODES), lambda i: (0, 0)),
            pl.BlockSpec((_NUM_CODES, _CODE_DIM), lambda i: (0, 0)),
            pl.BlockSpec((_BR, _NUM_CODES), lambda i: (i, 0)),
        ],
        out_specs=[
            pl.BlockSpec((_BR, _NUM_CODES), lambda i: (i, 0)),
            pl.BlockSpec((_BR, _CODE_DIM), lambda i: (i, 0)),
            pl.BlockSpec((_BR, _CODE_DIM), lambda i: (i, 0)),
            pl.BlockSpec((_BR, _CODE_DIM), lambda i: (i, 0)),
            pl.BlockSpec((1, 1), lambda i: (0, 0)),
        ],
        out_shape=[
            jax.ShapeDtypeStruct((_B, _NUM_CODES), jnp.float32),
            jax.ShapeDtypeStruct((_B, _CODE_DIM), jnp.float32),
            jax.ShapeDtypeStruct((_B, _CODE_DIM), jnp.float32),
            jax.ShapeDtypeStruct((_B, _CODE_DIM), jnp.float32),
            jax.ShapeDtypeStruct((1, 1), jnp.float32),
        ],
        compiler_params=pltpu.CompilerParams(
            dimension_semantics=("arbitrary",),
        ),
    )(h, cbt, codebook, gumbel)
    loss = loss_acc[0, 0] * ((1.0 + _BETA) / (_B * _CODE_DIM))
    return (q, ct, ch, cq, loss)


# speedup vs baseline: 1.0319x; 1.0319x over previous
import functools

import jax
import jax.numpy as jnp
from jax.experimental import pallas as pl
from jax.experimental.pallas import tpu as pltpu

_NUM_CODES = 8192
_CODE_DIM = 32
_BETA = 0.25
_B = 8192
_BR = 256
_NB = _B // _BR


def _vq_body(h_ref, cbt_ref, cb_ref, g_ref,
             q_ref, ct_ref, ch_ref, cq_ref, loss_ref):
    h = h_ref[...]
    cbt2 = cbt_ref[...]
    cb_sq = 0.25 * jnp.sum(cbt2 * cbt2, axis=0, keepdims=True)
    h_sq = jnp.sum(h * h, axis=1, keepdims=True)
    prod2 = jnp.dot(h, cbt2, preferred_element_type=jnp.float32)
    dist = (h_sq + cb_sq) - prod2
    x = g_ref[...] - dist
    m = jnp.max(x, axis=1, keepdims=True)
    e = jnp.exp(x - m)
    s = jnp.sum(e, axis=1, keepdims=True)
    inv = 1.0 / s
    q = e * inv
    q_ref[...] = q

    cb16 = cb_ref[...].astype(jnp.bfloat16)
    ct = jnp.dot(q.astype(jnp.bfloat16), cb16,
                 preferred_element_type=jnp.float32)
    ct_ref[...] = ct

    iota = jax.lax.broadcasted_iota(jnp.int32, (_BR, _NUM_CODES), 1)
    idx = jnp.min(jnp.where(x == m, iota, _NUM_CODES), axis=1, keepdims=True)
    onehot = (iota == idx).astype(jnp.bfloat16)
    ch = jnp.dot(onehot, cb16, preferred_element_type=jnp.float32)
    ch_ref[...] = ch
    cq_ref[...] = ct + (ch - ct)

    d = h - ch
    part = jnp.sum(d * d, axis=(0, 1), keepdims=True)
    loss_ref[...] = jnp.broadcast_to(part[None], (1, 1, 128))


@jax.jit
def kernel(h, codebook, gumbel):
    cbt2 = 2.0 * codebook.T
    q, ct, ch, cq, loss_acc = pl.pallas_call(
        _vq_body,
        grid=(_NB,),
        in_specs=[
            pl.BlockSpec((_BR, _CODE_DIM), lambda i: (i, 0)),
            pl.BlockSpec((_CODE_DIM, _NUM_CODES), lambda i: (0, 0)),
            pl.BlockSpec((_NUM_CODES, _CODE_DIM), lambda i: (0, 0)),
            pl.BlockSpec((_BR, _NUM_CODES), lambda i: (i, 0)),
        ],
        out_specs=[
            pl.BlockSpec((_BR, _NUM_CODES), lambda i: (i, 0)),
            pl.BlockSpec((_BR, _CODE_DIM), lambda i: (i, 0)),
            pl.BlockSpec((_BR, _CODE_DIM), lambda i: (i, 0)),
            pl.BlockSpec((_BR, _CODE_DIM), lambda i: (i, 0)),
            pl.BlockSpec((1, 1, 128), lambda i: (i, 0, 0)),
        ],
        out_shape=[
            jax.ShapeDtypeStruct((_B, _NUM_CODES), jnp.float32),
            jax.ShapeDtypeStruct((_B, _CODE_DIM), jnp.float32),
            jax.ShapeDtypeStruct((_B, _CODE_DIM), jnp.float32),
            jax.ShapeDtypeStruct((_B, _CODE_DIM), jnp.float32),
            jax.ShapeDtypeStruct((_NB, 1, 128), jnp.float32),
        ],
        compiler_params=pltpu.CompilerParams(
            dimension_semantics=("parallel",),
        ),
    )(h, cbt2, codebook, gumbel)
    loss = jnp.sum(loss_acc[:, 0, 0]) * ((1.0 + _BETA) / (_B * _CODE_DIM))
    return (q, ct, ch, cq, loss)
